# Initial kernel scaffold; baseline (speedup 1.0000x reference)
import jax
import jax.numpy as jnp
from jax.experimental import pallas as pl


def kernel(nfeats, efeats, W_msg_w, W_msg_b, W_apply_w, W_apply_b, edge_index):
    raise NotImplementedError("write your pallas kernel here")


# speedup vs baseline: 6.0056x; 6.0056x over previous
import functools

import jax
import jax.numpy as jnp
from jax import lax
from jax.experimental import pallas as pl
from jax.experimental.pallas import tpu as pltpu
from jax.experimental.pallas import tpu_sc as plsc

N = 10000
E = 320000
DIN = 128
DE = 16
DOUT = 128

NC = 2
NS = 16
NW = NC * NS
LANES = 16

EPW = E // NW
CH = 80
NCHUNK = EPW // CH
NPAIR = NCHUNK // 2
ROWS_PT = 624
TAIL_R = N - NS * ROWS_PT

_mesh = plsc.VectorSubcoreMesh(core_axis_name="c", subcore_axis_name="s")
_sc_params = pltpu.CompilerParams(use_tc_tiling_on_sc=False)


@functools.partial(
    pl.kernel,
    out_type=(
        jax.ShapeDtypeStruct((NC * N, DIN), jnp.float32),
        jax.ShapeDtypeStruct((NC * N, DE), jnp.float32),
        jax.ShapeDtypeStruct((NC * N, LANES), jnp.float32),
    ),
    mesh=_mesh,
    compiler_params=_sc_params,
    scratch_types=[
        pltpu.VMEM((CH,), jnp.int32),
        pltpu.VMEM((CH,), jnp.int32),
        pltpu.VMEM((CH,), jnp.int32),
        pltpu.VMEM((CH,), jnp.int32),
        pltpu.VMEM((CH, DIN), jnp.float32),
        pltpu.VMEM((CH, DIN), jnp.float32),
        pltpu.VMEM((CH, DE), jnp.float32),
        pltpu.VMEM((CH, DE), jnp.float32),
        pltpu.VMEM((CH, LANES), jnp.float32),
        pltpu.VMEM_SHARED((N, DIN), jnp.float32),
        pltpu.VMEM_SHARED((N, DE), jnp.float32),
        pltpu.VMEM_SHARED((N, LANES), jnp.float32),
        pltpu.SemaphoreType.DMA,
        pltpu.SemaphoreType.DMA,
        pltpu.SemaphoreType.DMA,
        pltpu.SemaphoreType.DMA,
        pltpu.SemaphoreType.DMA,
        pltpu.SemaphoreType.DMA,
        pltpu.SemaphoreType.DMA,
        pltpu.SemaphoreType.DMA,
        pltpu.SemaphoreType.DMA,
        pltpu.SemaphoreType.DMA,
    ],
)
def _sc_agg(nfeats_hbm, efeats_hbm, src_hbm, dst_hbm,
            Ap_hbm, Ep_hbm, dp_hbm,
            srcA, srcB, dstA, dstB, rows0, rows1, ef0, ef1, ones_v,
            A_sh, E_sh, d_sh,
            si0, si1, sd0, sd1, sg0, sg1, se0, se1, ss0, ss1):
    c = lax.axis_index("c")
    s = lax.axis_index("s")
    wid = s * NC + c

    def zrow(r, carry):
        for j in range(DIN // LANES):
            rows0[r, pl.ds(j * LANES, LANES)] = jnp.zeros((LANES,), jnp.float32)
        ef0[r, :] = jnp.zeros((LANES,), jnp.float32)
        return carry

    lax.fori_loop(0, CH, zrow, 0)

    r0 = s * ROWS_PT
    for k in range(7):
        pltpu.sync_copy(rows0, A_sh.at[pl.ds(r0 + k * CH, CH)])
        pltpu.sync_copy(ef0, E_sh.at[pl.ds(r0 + k * CH, CH)])
        pltpu.sync_copy(ef0, d_sh.at[pl.ds(r0 + k * CH, CH)])
    rem = ROWS_PT - 7 * CH
    pltpu.sync_copy(rows0.at[pl.ds(0, rem)], A_sh.at[pl.ds(r0 + 7 * CH, rem)])
    pltpu.sync_copy(ef0.at[pl.ds(0, rem)], E_sh.at[pl.ds(r0 + 7 * CH, rem)])
    pltpu.sync_copy(ef0.at[pl.ds(0, rem)], d_sh.at[pl.ds(r0 + 7 * CH, rem)])

    @pl.when(s == 0)
    def _():
        pltpu.sync_copy(rows0.at[pl.ds(0, TAIL_R)],
                        A_sh.at[pl.ds(NS * ROWS_PT, TAIL_R)])
        pltpu.sync_copy(ef0.at[pl.ds(0, TAIL_R)],
                        E_sh.at[pl.ds(NS * ROWS_PT, TAIL_R)])
        pltpu.sync_copy(ef0.at[pl.ds(0, TAIL_R)],
                        d_sh.at[pl.ds(NS * ROWS_PT, TAIL_R)])

    for r in range(CH):
        ones_v[r, :] = jnp.ones((LANES,), jnp.float32)

    plsc.subcore_barrier()

    base0 = wid * EPW

    def do_chunk_pair(j, carry):
        ba = base0 + (2 * j) * CH
        bb = ba + CH
        ia = pltpu.async_copy(src_hbm.at[pl.ds(ba, CH)], srcA, si0)
        da = pltpu.async_copy(dst_hbm.at[pl.ds(ba, CH)], dstA, sd0)
        ib = pltpu.async_copy(src_hbm.at[pl.ds(bb, CH)], srcB, si1)
        db = pltpu.async_copy(dst_hbm.at[pl.ds(bb, CH)], dstB, sd1)
        ia.wait()
        ga = pltpu.async_copy(nfeats_hbm.at[srcA], rows0, sg0)
        ea = pltpu.async_copy(efeats_hbm.at[pl.ds(ba, CH)], ef0, se0)
        ib.wait()
        gb = pltpu.async_copy(nfeats_hbm.at[srcB], rows1, sg1)
        eb = pltpu.async_copy(efeats_hbm.at[pl.ds(bb, CH)], ef1, se1)
        ga.wait()
        ea.wait()
        da.wait()
        s0a = pltpu.async_copy(rows0, A_sh.at[dstA], ss0, add=True)
        s1a = pltpu.async_copy(ef0, E_sh.at[dstA], ss0, add=True)
        s2a = pltpu.async_copy(ones_v, d_sh.at[dstA], ss0, add=True)
        gb.wait()
        eb.wait()
        db.wait()
        s0b = pltpu.async_copy(rows1, A_sh.at[dstB], ss1, add=True)
        s1b = pltpu.async_copy(ef1, E_sh.at[dstB], ss1, add=True)
        s2b = pltpu.async_copy(ones_v, d_sh.at[dstB], ss1, add=True)
        s0a.wait()
        s1a.wait()
        s2a.wait()
        s0b.wait()
        s1b.wait()
        s2b.wait()
        return carry

    lax.fori_loop(0, NPAIR, do_chunk_pair, 0)

    bt = base0 + (NCHUNK - 1) * CH
    it = pltpu.async_copy(src_hbm.at[pl.ds(bt, CH)], srcA, si0)
    dt = pltpu.async_copy(dst_hbm.at[pl.ds(bt, CH)], dstA, sd0)
    it.wait()
    gt = pltpu.async_copy(nfeats_hbm.at[srcA], rows0, sg0)
    et = pltpu.async_copy(efeats_hbm.at[pl.ds(bt, CH)], ef0, se0)
    gt.wait()
    et.wait()
    dt.wait()
    s0t = pltpu.async_copy(rows0, A_sh.at[dstA], ss0, add=True)
    s1t = pltpu.async_copy(ef0, E_sh.at[dstA], ss0, add=True)
    s2t = pltpu.async_copy(ones_v, d_sh.at[dstA], ss0, add=True)
    s0t.wait()
    s1t.wait()
    s2t.wait()

    plsc.subcore_barrier()

    o0 = c * N + r0

    def out_chunk(roff, nrows):
        pltpu.sync_copy(A_sh.at[pl.ds(r0 + roff, nrows)], rows0.at[pl.ds(0, nrows)])
        pltpu.sync_copy(rows0.at[pl.ds(0, nrows)], Ap_hbm.at[pl.ds(o0 + roff, nrows)])
        pltpu.sync_copy(E_sh.at[pl.ds(r0 + roff, nrows)], ef0.at[pl.ds(0, nrows)])
        pltpu.sync_copy(ef0.at[pl.ds(0, nrows)], Ep_hbm.at[pl.ds(o0 + roff, nrows)])
        pltpu.sync_copy(d_sh.at[pl.ds(r0 + roff, nrows)], ones_v.at[pl.ds(0, nrows)])
        pltpu.sync_copy(ones_v.at[pl.ds(0, nrows)], dp_hbm.at[pl.ds(o0 + roff, nrows)])

    for k in range(7):
        out_chunk(k * CH, CH)
    out_chunk(7 * CH, rem)

    @pl.when(s == 0)
    def _():
        t0 = NS * ROWS_PT
        pltpu.sync_copy(A_sh.at[pl.ds(t0, TAIL_R)], rows0.at[pl.ds(0, TAIL_R)])
        pltpu.sync_copy(rows0.at[pl.ds(0, TAIL_R)], Ap_hbm.at[pl.ds(c * N + t0, TAIL_R)])
        pltpu.sync_copy(E_sh.at[pl.ds(t0, TAIL_R)], ef0.at[pl.ds(0, TAIL_R)])
        pltpu.sync_copy(ef0.at[pl.ds(0, TAIL_R)], Ep_hbm.at[pl.ds(c * N + t0, TAIL_R)])
        pltpu.sync_copy(d_sh.at[pl.ds(t0, TAIL_R)], ones_v.at[pl.ds(0, TAIL_R)])
        pltpu.sync_copy(ones_v.at[pl.ds(0, TAIL_R)], dp_hbm.at[pl.ds(c * N + t0, TAIL_R)])


_R = 1000


def _tc_dense_body(a0, a1, e0, e1, d0, d1, nf, W1, W2, bm, Wat, Wab, ba, out):
    a = a0[...] + a1[...]
    e = e0[...] + e1[...]
    d = d0[...] + d1[...]
    msum = (jnp.dot(a, W1[...], preferred_element_type=jnp.float32)
            + jnp.dot(e, W2[...], preferred_element_type=jnp.float32)
            + d * bm[...])
    h_neigh = msum * (1.0 / jnp.maximum(d, 1.0))
    pre = (jnp.dot(nf[...], Wat[...], preferred_element_type=jnp.float32)
           + jnp.dot(h_neigh, Wab[...], preferred_element_type=jnp.float32)
           + ba[...])
    out[...] = jnp.maximum(pre, 0.0)


def _tc_dense(a0, a1, e0, e1, d0, d1, nf, W1, W2, bm, Wat, Wab, ba):
    row = lambda w: pl.BlockSpec((_R, w), lambda i: (i, 0))
    full = lambda s: pl.BlockSpec(s, lambda i: (0, 0))
    return pl.pallas_call(
        _tc_dense_body,
        grid=(N // _R,),
        in_specs=[
            row(DIN), row(DIN), row(DE), row(DE), row(1), row(1), row(DIN),
            full((DIN, DOUT)), full((DE, DOUT)), full((1, DOUT)),
            full((DIN, DOUT)), full((DOUT, DOUT)), full((1, DOUT)),
        ],
        out_specs=row(DOUT),
        out_shape=jax.ShapeDtypeStruct((N, DOUT), jnp.float32),
    )(a0, a1, e0, e1, d0, d1, nf, W1, W2, bm, Wat, Wab, ba)


@functools.partial(
    pl.kernel,
    out_type=jax.ShapeDtypeStruct((E, DOUT), jnp.float32),
    mesh=_mesh,
    compiler_params=_sc_params,
    scratch_types=[
        pltpu.VMEM((NCHUNK, CH), jnp.int32),
        pltpu.VMEM((NCHUNK, CH), jnp.int32),
        pltpu.VMEM((CH, DOUT), jnp.float32),
        pltpu.VMEM((CH, DOUT), jnp.float32),
        pltpu.VMEM((CH, DOUT), jnp.float32),
        pltpu.VMEM((CH, DOUT), jnp.float32),
        pltpu.VMEM((CH, DOUT), jnp.float32),
        pltpu.VMEM((CH, DOUT), jnp.float32),
        pltpu.SemaphoreType.DMA,
        pltpu.SemaphoreType.DMA,
        pltpu.SemaphoreType.DMA,
        pltpu.SemaphoreType.DMA,
        pltpu.SemaphoreType.DMA,
        pltpu.SemaphoreType.DMA,
    ],
)
def _sc_edge(nh_hbm, src3_hbm, dst3_hbm, out_hbm,
             srcs, dsts, bufA0, bufA1, bufB0, bufB1, bufO0, bufO1,
             sa0, sa1, sb0, sb1, sw0, sw1):
    c = lax.axis_index("c")
    s = lax.axis_index("s")
    wid = s * NC + c
    base0 = wid * EPW

    pltpu.sync_copy(src3_hbm.at[wid], srcs)
    pltpu.sync_copy(dst3_hbm.at[wid], dsts)

    def average(bufA, bufB, bufO):
        def row(r, rc):
            for j in range(DOUT // LANES):
                sl = pl.ds(j * LANES, LANES)
                bufO[r, sl] = (bufA[r, sl] + bufB[r, sl]) * 0.5
            return rc

        lax.fori_loop(0, CH, row, 0)

    def do_chunk_pair(j, carry):
        a = 2 * j
        b = 2 * j + 1
        gAa = pltpu.async_copy(nh_hbm.at[srcs.at[a]], bufA0, sa0)
        gBa = pltpu.async_copy(nh_hbm.at[dsts.at[a]], bufB0, sb0)
        gAb = pltpu.async_copy(nh_hbm.at[srcs.at[b]], bufA1, sa1)
        gBb = pltpu.async_copy(nh_hbm.at[dsts.at[b]], bufB1, sb1)
        gAa.wait()
        gBa.wait()
        average(bufA0, bufB0, bufO0)
        wa = pltpu.async_copy(bufO0, out_hbm.at[pl.ds(base0 + a * CH, CH)], sw0)
        gAb.wait()
        gBb.wait()
        average(bufA1, bufB1, bufO1)
        wb = pltpu.async_copy(bufO1, out_hbm.at[pl.ds(base0 + b * CH, CH)], sw1)
        wa.wait()
        wb.wait()
        return carry

    lax.fori_loop(0, NPAIR, do_chunk_pair, 0)

    t = NCHUNK - 1
    gAt = pltpu.async_copy(nh_hbm.at[srcs.at[t]], bufA0, sa0)
    gBt = pltpu.async_copy(nh_hbm.at[dsts.at[t]], bufB0, sb0)
    gAt.wait()
    gBt.wait()
    average(bufA0, bufB0, bufO0)
    pltpu.sync_copy(bufO0, out_hbm.at[pl.ds(base0 + t * CH, CH)])


def kernel(nfeats, efeats, W_msg_w, W_msg_b, W_apply_w, W_apply_b, edge_index):
    src = edge_index[0]
    dst = edge_index[1]
    src3 = src.reshape(NW, NCHUNK, CH)
    dst3 = dst.reshape(NW, NCHUNK, CH)

    Ap, Ep, dp = _sc_agg(nfeats, efeats, src, dst)

    new_h = _tc_dense(
        Ap[:N], Ap[N:], Ep[:N], Ep[N:],
        dp[:N, 0:1], dp[N:, 0:1], nfeats,
        W_msg_w[:DIN], W_msg_w[DIN:], W_msg_b[None, :],
        W_apply_w[:DIN], W_apply_w[DIN:], W_apply_b[None, :],
    )

    new_e = _sc_edge(new_h, src3, dst3)
    return (new_h, new_e)
